# baseline (speedup 1.0000x reference)
import functools

import jax
import jax.numpy as jnp
from jax.experimental import pallas as pl
from jax.experimental.pallas import tpu as pltpu


def _angle(hi, hr):
    m = (hr * hr + hi * hi) > 1e-12
    hr_s = jnp.where(m, hr, 1.0)
    hi_s = jnp.where(m, hi, 0.0)
    return jnp.where(m, jnp.arctan2(hi_s, hr_s), 0.0)


def _fwd_body(*refs, n_layers, bn, nblk, h_dim, n_cls, nstream):
    adj_refs = refs[:nstream]
    (x_ref, w0_ref, b0_ref, m_ref, wc_ref, mo_ref, bo_ref, lab_ref, msk_ref,
     hr_ref, hi_ref, ang_ref, nrm_ref, or_ref, oi_ref, sep_ref, sup_ref,
     h_cur, h_nxt, cm_acc, cnt_acc, sup_acc, msk_acc, sep_sum) = refs[nstream:]
    l = pl.program_id(0)
    i = pl.program_id(1)

    @pl.when((l == 0) & (i == 0))
    def _init():
        h_cur[...] = jnp.maximum(
            jnp.dot(x_ref[...], w0_ref[...],
                    preferred_element_type=jnp.float32) + b0_ref[...], 0.0)
        sup_acc[...] = jnp.zeros_like(sup_acc)
        msk_acc[...] = jnp.zeros_like(msk_acc)
        sep_sum[...] = jnp.zeros_like(sep_sum)

    @pl.when(i == 0)
    def _layer_init():
        cm_acc[...] = jnp.zeros_like(cm_acc)

    @pl.when((l == 0) & (i == 0))
    def _cnt_init():
        cnt_acc[...] = jnp.zeros_like(cnt_acc)

    a = jnp.concatenate(
        [jnp.dot(r[...], h_cur[...], preferred_element_type=jnp.float32)
         for r in adj_refs], axis=0)
    n = jnp.dot(a, m_ref[0], preferred_element_type=jnp.float32)
    nr = n[:, :h_dim]
    ni = n[:, h_dim:]

    logits = jnp.dot(n, wc_ref[0], preferred_element_type=jnp.float32)
    mx = jnp.max(logits, axis=-1, keepdims=True)
    lse = mx + jnp.log(jnp.sum(jnp.exp(logits - mx), axis=-1, keepdims=True))
    logp = logits - lse
    lab = lab_ref[pl.ds(i * bn, bn), :]
    classes = jax.lax.broadcasted_iota(jnp.int32, (bn, n_cls), 1)
    onehot = (lab == classes).astype(jnp.float32)
    maskf = msk_ref[pl.ds(i * bn, bn), :]
    picked = jnp.sum(onehot * logp, axis=-1, keepdims=True)
    sup_acc[...] = sup_acc[...] + jnp.sum(picked * maskf)
    msk_acc[...] = msk_acc[...] + jnp.sum(maskf)

    ang = _angle(ni, nr)
    cm_acc[...] = cm_acc[...] + jax.lax.dot_general(
        onehot, ang, (((0,), (0,)), ((), ())),
        preferred_element_type=jnp.float32)
    @pl.when(l == 0)
    def _count():
        cnt_acc[...] = cnt_acc[...] + jax.lax.dot_general(
            onehot, jnp.ones((bn, h_dim), jnp.float32),
            (((0,), (0,)), ((), ())), preferred_element_type=jnp.float32)

    h_nxt[pl.ds(i * bn, bn), :] = jnp.maximum(n, 0.0)

    @pl.when(l == n_layers - 1)
    def _readout():
        hr = jnp.maximum(nr, 0.0)
        hi = jnp.maximum(ni, 0.0)
        hr_ref[...] = hr
        hi_ref[...] = hi
        ang_ref[...] = _angle(hi, hr)
        nrm_ref[...] = jnp.sqrt(hr * hr + hi * hi + 1e-12)
        oo = jnp.dot(jnp.maximum(n, 0.0), mo_ref[...],
                     preferred_element_type=jnp.float32) + bo_ref[...]
        or_ref[...] = oo[:, :n_cls]
        oi_ref[...] = oo[:, n_cls:]

    @pl.when(i == nblk - 1)
    def _layer_end():
        cm = cm_acc[...] / (cnt_acc[...] + 1e-8)
        sc = jnp.sum(jnp.cos(cm), axis=0)
        ss = jnp.sum(jnp.sin(cm), axis=0)
        tot = jnp.sum(sc * sc + ss * ss) - float(n_cls * h_dim)
        sep_sum[...] = sep_sum[...] + tot / float(
            (n_cls * n_cls - n_cls) * h_dim)

        @pl.when(l < n_layers - 1)
        def _promote():
            h_cur[...] = h_nxt[...]

        @pl.when(l == n_layers - 1)
        def _emit():
            sep_ref[...] = sep_sum[...]
            sup = -sup_acc[0, 0] / (msk_acc[0, 0] / float(n_layers) + 1e-8)
            sup_ref[...] = sup * jnp.ones((1, 1), jnp.float32)


def kernel(x, adj, Wr0, Wi0, br0, bi0, conv_Wr, conv_Wi, conv_Wc,
           Wr1, Wi1, br1, bi1, labels, train_mask):
    n_nodes = adj.shape[0]
    f_in = x.shape[1]
    h_dim = Wr0.shape[1]
    n_cls = conv_Wc.shape[-1]
    n_layers = conv_Wr.shape[0]
    bn = next(b for b in (200, 400, 100, 40, 8, 4, 2, 1) if n_nodes % b == 0)
    nblk = n_nodes // bn
    nstream = next((k for k in (2, 1) if bn % k == 0 and (bn // k) % 8 == 0),
                   1)
    br = bn // nstream

    w0 = jnp.concatenate([Wr0, Wi0], axis=1)
    b0 = jnp.concatenate([br0, bi0])[None, :]
    ms = jnp.concatenate(
        [jnp.concatenate([conv_Wr, conv_Wi], axis=2),
         jnp.concatenate([-conv_Wi, conv_Wr], axis=2)], axis=1)
    mo = jnp.block([[Wr1, Wi1], [-Wi1, Wr1]])
    bo = jnp.concatenate([br1, bi1])[None, :]
    lab2 = labels.reshape(n_nodes, 1)
    msk2 = train_mask.astype(jnp.float32).reshape(n_nodes, 1)

    adj_specs = [
        pl.BlockSpec((br, n_nodes),
                     functools.partial(lambda s, l, i: (nstream * i + s, 0), s))
        for s in range(nstream)
    ]
    const2 = lambda l, i: (0, 0)
    row2 = lambda l, i: (i, 0)
    outrow = lambda l, i: (jnp.where(l == n_layers - 1, i, 0), 0)
    in_specs = adj_specs + [
        pl.BlockSpec((n_nodes, f_in), const2),
        pl.BlockSpec((f_in, 2 * h_dim), const2),
        pl.BlockSpec((1, 2 * h_dim), const2),
        pl.BlockSpec((1, 2 * h_dim, 2 * h_dim), lambda l, i: (l, 0, 0)),
        pl.BlockSpec((1, 2 * h_dim, n_cls), lambda l, i: (l, 0, 0)),
        pl.BlockSpec((2 * h_dim, 2 * n_cls), const2),
        pl.BlockSpec((1, 2 * n_cls), const2),
        pl.BlockSpec((n_nodes, 1), const2),
        pl.BlockSpec((n_nodes, 1), const2),
    ]
    scalar_spec = pl.BlockSpec((1, 1), const2)
    scalar_shape = jax.ShapeDtypeStruct((1, 1), jnp.float32)
    nh_shape = jax.ShapeDtypeStruct((n_nodes, h_dim), jnp.float32)
    body = functools.partial(_fwd_body, n_layers=n_layers, bn=bn, nblk=nblk,
                             h_dim=h_dim, n_cls=n_cls, nstream=nstream)
    hr, hi, hang, hnrm, out_r, out_i, sep, sup = pl.pallas_call(
        body,
        grid=(n_layers, nblk),
        in_specs=in_specs,
        out_specs=[
            pl.BlockSpec((bn, h_dim), outrow),
            pl.BlockSpec((bn, h_dim), outrow),
            pl.BlockSpec((bn, h_dim), outrow),
            pl.BlockSpec((bn, h_dim), outrow),
            pl.BlockSpec((bn, n_cls), outrow),
            pl.BlockSpec((bn, n_cls), outrow),
            scalar_spec, scalar_spec,
        ],
        out_shape=[
            nh_shape, nh_shape, nh_shape, nh_shape,
            jax.ShapeDtypeStruct((n_nodes, n_cls), jnp.float32),
            jax.ShapeDtypeStruct((n_nodes, n_cls), jnp.float32),
            scalar_shape, scalar_shape,
        ],
        compiler_params=pltpu.CompilerParams(
            vmem_limit_bytes=62 * 1024 * 1024),
        scratch_shapes=[
            pltpu.VMEM((n_nodes, 2 * h_dim), jnp.float32),
            pltpu.VMEM((n_nodes, 2 * h_dim), jnp.float32),
            pltpu.VMEM((n_cls, h_dim), jnp.float32),
            pltpu.VMEM((n_cls, h_dim), jnp.float32),
            pltpu.VMEM((1, 1), jnp.float32),
            pltpu.VMEM((1, 1), jnp.float32),
            pltpu.VMEM((1, 1), jnp.float32),
        ],
    )(*([adj] * nstream), x, w0, b0, ms, conv_Wc, mo, bo, lab2, msk2)

    return (hr, hi, out_r, out_i, hang, hnrm, sep[0, 0], sup[0, 0])


# speedup vs baseline: 1.1399x; 1.1399x over previous
import functools

import jax
import jax.numpy as jnp
from jax.experimental import pallas as pl
from jax.experimental.pallas import tpu as pltpu


def _angle(hi, hr):
    m = (hr * hr + hi * hi) > 1e-12
    hr_s = jnp.where(m, hr, 1.0)
    hi_s = jnp.where(m, hi, 0.0)
    return jnp.where(m, jnp.arctan2(hi_s, hr_s), 0.0)


def _fwd_body(*refs, n_layers, bn, nblk, h_dim, n_cls, nstream):
    adj_refs = refs[:nstream]
    (x_ref, w0_ref, b0_ref, m_ref, wc_ref, mo_ref, bo_ref, lab_ref, msk_ref,
     hr_ref, hi_ref, ang_ref, nrm_ref, or_ref, oi_ref, sep_ref, sup_ref,
     h_cur, h_nxt, cm_acc, cnt_acc, sup_acc, msk_acc, sep_sum) = refs[nstream:]
    l = pl.program_id(0)
    i = pl.program_id(1)

    @pl.when((l == 0) & (i == 0))
    def _init():
        h_cur[...] = jnp.maximum(
            jnp.dot(x_ref[...], w0_ref[...],
                    preferred_element_type=jnp.float32) + b0_ref[...], 0.0)
        sup_acc[...] = jnp.zeros_like(sup_acc)
        msk_acc[...] = jnp.zeros_like(msk_acc)
        sep_sum[...] = jnp.zeros_like(sep_sum)

    @pl.when(i == 0)
    def _layer_init():
        cm_acc[...] = jnp.zeros_like(cm_acc)

    @pl.when((l == 0) & (i == 0))
    def _cnt_init():
        cnt_acc[...] = jnp.zeros_like(cnt_acc)

    a = jnp.concatenate(
        [jnp.dot(r[...], h_cur[...], preferred_element_type=jnp.float32)
         for r in adj_refs], axis=0)
    n = jnp.dot(a, m_ref[0], preferred_element_type=jnp.float32)
    nr = n[:, :h_dim]
    ni = n[:, h_dim:]

    logits = jnp.dot(n, wc_ref[0], preferred_element_type=jnp.float32)
    mx = jnp.max(logits, axis=-1, keepdims=True)
    lse = mx + jnp.log(jnp.sum(jnp.exp(logits - mx), axis=-1, keepdims=True))
    logp = logits - lse
    lab = lab_ref[pl.ds(i * bn, bn), :]
    classes = jax.lax.broadcasted_iota(jnp.int32, (bn, n_cls), 1)
    onehot = (lab == classes).astype(jnp.float32)
    maskf = msk_ref[pl.ds(i * bn, bn), :]
    picked = jnp.sum(onehot * logp, axis=-1, keepdims=True)
    sup_acc[...] = sup_acc[...] + jnp.sum(picked * maskf)
    msk_acc[...] = msk_acc[...] + jnp.sum(maskf)

    ang = _angle(ni, nr)
    cm_acc[...] = cm_acc[...] + jax.lax.dot_general(
        onehot, ang, (((0,), (0,)), ((), ())),
        preferred_element_type=jnp.float32)
    @pl.when(l == 0)
    def _count():
        cnt_acc[...] = cnt_acc[...] + jax.lax.dot_general(
            onehot, jnp.ones((bn, h_dim), jnp.float32),
            (((0,), (0,)), ((), ())), preferred_element_type=jnp.float32)

    h_nxt[pl.ds(i * bn, bn), :] = jnp.maximum(n, 0.0)

    @pl.when(l == n_layers - 1)
    def _readout():
        hr = jnp.maximum(nr, 0.0)
        hi = jnp.maximum(ni, 0.0)
        hr_ref[...] = hr
        hi_ref[...] = hi
        ang_ref[...] = _angle(hi, hr)
        nrm_ref[...] = jnp.sqrt(hr * hr + hi * hi + 1e-12)
        oo = jnp.dot(jnp.maximum(n, 0.0), mo_ref[...],
                     preferred_element_type=jnp.float32) + bo_ref[...]
        or_ref[...] = oo[:, :n_cls]
        oi_ref[...] = oo[:, n_cls:]

    @pl.when(i == nblk - 1)
    def _layer_end():
        cm = cm_acc[...] / (cnt_acc[...] + 1e-8)
        sc = jnp.sum(jnp.cos(cm), axis=0)
        ss = jnp.sum(jnp.sin(cm), axis=0)
        tot = jnp.sum(sc * sc + ss * ss) - float(n_cls * h_dim)
        sep_sum[...] = sep_sum[...] + tot / float(
            (n_cls * n_cls - n_cls) * h_dim)

        @pl.when(l < n_layers - 1)
        def _promote():
            h_cur[...] = h_nxt[...]

        @pl.when(l == n_layers - 1)
        def _emit():
            sep_ref[...] = sep_sum[...]
            sup = -sup_acc[0, 0] / (msk_acc[0, 0] / float(n_layers) + 1e-8)
            sup_ref[...] = sup * jnp.ones((1, 1), jnp.float32)


def kernel(x, adj, Wr0, Wi0, br0, bi0, conv_Wr, conv_Wi, conv_Wc,
           Wr1, Wi1, br1, bi1, labels, train_mask):
    n_nodes = adj.shape[0]
    f_in = x.shape[1]
    h_dim = Wr0.shape[1]
    n_cls = conv_Wc.shape[-1]
    n_layers = conv_Wr.shape[0]
    bn = next(b for b in (400, 200, 100, 40, 8, 4, 2, 1) if n_nodes % b == 0)
    nblk = n_nodes // bn
    nstream = next((k for k in (2, 1) if bn % k == 0 and (bn // k) % 8 == 0),
                   1)
    br = bn // nstream

    w0 = jnp.concatenate([Wr0, Wi0], axis=1)
    b0 = jnp.concatenate([br0, bi0])[None, :]
    ms = jnp.concatenate(
        [jnp.concatenate([conv_Wr, conv_Wi], axis=2),
         jnp.concatenate([-conv_Wi, conv_Wr], axis=2)], axis=1)
    mo = jnp.block([[Wr1, Wi1], [-Wi1, Wr1]])
    bo = jnp.concatenate([br1, bi1])[None, :]
    lab2 = labels.reshape(n_nodes, 1)
    msk2 = train_mask.astype(jnp.float32).reshape(n_nodes, 1)

    adj_specs = [
        pl.BlockSpec((br, n_nodes),
                     functools.partial(lambda s, l, i: (nstream * i + s, 0), s))
        for s in range(nstream)
    ]
    const2 = lambda l, i: (0, 0)
    row2 = lambda l, i: (i, 0)
    outrow = lambda l, i: (jnp.where(l == n_layers - 1, i, 0), 0)
    in_specs = adj_specs + [
        pl.BlockSpec((n_nodes, f_in), const2),
        pl.BlockSpec((f_in, 2 * h_dim), const2),
        pl.BlockSpec((1, 2 * h_dim), const2),
        pl.BlockSpec((1, 2 * h_dim, 2 * h_dim), lambda l, i: (l, 0, 0)),
        pl.BlockSpec((1, 2 * h_dim, n_cls), lambda l, i: (l, 0, 0)),
        pl.BlockSpec((2 * h_dim, 2 * n_cls), const2),
        pl.BlockSpec((1, 2 * n_cls), const2),
        pl.BlockSpec((n_nodes, 1), const2),
        pl.BlockSpec((n_nodes, 1), const2),
    ]
    scalar_spec = pl.BlockSpec((1, 1), const2)
    scalar_shape = jax.ShapeDtypeStruct((1, 1), jnp.float32)
    nh_shape = jax.ShapeDtypeStruct((n_nodes, h_dim), jnp.float32)
    body = functools.partial(_fwd_body, n_layers=n_layers, bn=bn, nblk=nblk,
                             h_dim=h_dim, n_cls=n_cls, nstream=nstream)
    hr, hi, hang, hnrm, out_r, out_i, sep, sup = pl.pallas_call(
        body,
        grid=(n_layers, nblk),
        in_specs=in_specs,
        out_specs=[
            pl.BlockSpec((bn, h_dim), outrow),
            pl.BlockSpec((bn, h_dim), outrow),
            pl.BlockSpec((bn, h_dim), outrow),
            pl.BlockSpec((bn, h_dim), outrow),
            pl.BlockSpec((bn, n_cls), outrow),
            pl.BlockSpec((bn, n_cls), outrow),
            scalar_spec, scalar_spec,
        ],
        out_shape=[
            nh_shape, nh_shape, nh_shape, nh_shape,
            jax.ShapeDtypeStruct((n_nodes, n_cls), jnp.float32),
            jax.ShapeDtypeStruct((n_nodes, n_cls), jnp.float32),
            scalar_shape, scalar_shape,
        ],
        compiler_params=pltpu.CompilerParams(
            vmem_limit_bytes=62 * 1024 * 1024),
        scratch_shapes=[
            pltpu.VMEM((n_nodes, 2 * h_dim), jnp.float32),
            pltpu.VMEM((n_nodes, 2 * h_dim), jnp.float32),
            pltpu.VMEM((n_cls, h_dim), jnp.float32),
            pltpu.VMEM((n_cls, h_dim), jnp.float32),
            pltpu.VMEM((1, 1), jnp.float32),
            pltpu.VMEM((1, 1), jnp.float32),
            pltpu.VMEM((1, 1), jnp.float32),
        ],
    )(*([adj] * nstream), x, w0, b0, ms, conv_Wc, mo, bo, lab2, msk2)

    return (hr, hi, out_r, out_i, hang, hnrm, sep[0, 0], sup[0, 0])


# speedup vs baseline: 1.1421x; 1.0019x over previous
import functools

import jax
import jax.numpy as jnp
from jax.experimental import pallas as pl
from jax.experimental.pallas import tpu as pltpu


def _angle(hi, hr):
    m = (hr * hr + hi * hi) > 1e-12
    hr_s = jnp.where(m, hr, 1.0)
    hi_s = jnp.where(m, hi, 0.0)
    return jnp.where(m, jnp.arctan2(hi_s, hr_s), 0.0)


def _fwd_body(*refs, n_layers, bn, nblk, h_dim, n_cls, nstream):
    adj_refs = refs[:nstream]
    (x_ref, w0_ref, b0_ref, m_ref, wc_ref, mo_ref, bo_ref, lab_ref, msk_ref,
     hr_ref, hi_ref, ang_ref, nrm_ref, or_ref, oi_ref, sep_ref, sup_ref,
     h_cur, h_nxt, cm_acc, cnt_acc, sup_acc, msk_acc, sep_sum) = refs[nstream:]
    t = pl.program_id(0)
    l = t // nblk
    i = t % nblk

    @pl.when((l == 0) & (i == 0))
    def _init():
        h_cur[...] = jnp.maximum(
            jnp.dot(x_ref[...], w0_ref[...],
                    preferred_element_type=jnp.float32) + b0_ref[...], 0.0)
        sup_acc[...] = jnp.zeros_like(sup_acc)
        msk_acc[...] = jnp.zeros_like(msk_acc)
        sep_sum[...] = jnp.zeros_like(sep_sum)

    @pl.when(i == 0)
    def _layer_init():
        cm_acc[...] = jnp.zeros_like(cm_acc)

    @pl.when((l == 0) & (i == 0))
    def _cnt_init():
        cnt_acc[...] = jnp.zeros_like(cnt_acc)

    a = jnp.concatenate(
        [jnp.dot(r[...], h_cur[...], preferred_element_type=jnp.float32)
         for r in adj_refs], axis=0)
    n = jnp.dot(a, m_ref[0], preferred_element_type=jnp.float32)
    nr = n[:, :h_dim]
    ni = n[:, h_dim:]

    logits = jnp.dot(n, wc_ref[0], preferred_element_type=jnp.float32)
    mx = jnp.max(logits, axis=-1, keepdims=True)
    lse = mx + jnp.log(jnp.sum(jnp.exp(logits - mx), axis=-1, keepdims=True))
    logp = logits - lse
    lab = lab_ref[pl.ds(i * bn, bn), :]
    classes = jax.lax.broadcasted_iota(jnp.int32, (bn, n_cls), 1)
    onehot = (lab == classes).astype(jnp.float32)
    maskf = msk_ref[pl.ds(i * bn, bn), :]
    picked = jnp.sum(onehot * logp, axis=-1, keepdims=True)
    sup_acc[...] = sup_acc[...] + jnp.sum(picked * maskf)
    msk_acc[...] = msk_acc[...] + jnp.sum(maskf)

    ang = _angle(ni, nr)
    cm_acc[...] = cm_acc[...] + jax.lax.dot_general(
        onehot, ang, (((0,), (0,)), ((), ())),
        preferred_element_type=jnp.float32)
    @pl.when(l == 0)
    def _count():
        cnt_acc[...] = cnt_acc[...] + jax.lax.dot_general(
            onehot, jnp.ones((bn, h_dim), jnp.float32),
            (((0,), (0,)), ((), ())), preferred_element_type=jnp.float32)

    h_nxt[pl.ds(i * bn, bn), :] = jnp.maximum(n, 0.0)

    @pl.when(l == n_layers - 1)
    def _readout():
        hr = jnp.maximum(nr, 0.0)
        hi = jnp.maximum(ni, 0.0)
        hr_ref[...] = hr
        hi_ref[...] = hi
        ang_ref[...] = _angle(hi, hr)
        nrm_ref[...] = jnp.sqrt(hr * hr + hi * hi + 1e-12)
        oo = jnp.dot(jnp.maximum(n, 0.0), mo_ref[...],
                     preferred_element_type=jnp.float32) + bo_ref[...]
        or_ref[...] = oo[:, :n_cls]
        oi_ref[...] = oo[:, n_cls:]

    @pl.when(i == nblk - 1)
    def _layer_end():
        cm = cm_acc[...] / (cnt_acc[...] + 1e-8)
        sc = jnp.sum(jnp.cos(cm), axis=0)
        ss = jnp.sum(jnp.sin(cm), axis=0)
        tot = jnp.sum(sc * sc + ss * ss) - float(n_cls * h_dim)
        sep_sum[...] = sep_sum[...] + tot / float(
            (n_cls * n_cls - n_cls) * h_dim)

        @pl.when(l < n_layers - 1)
        def _promote():
            h_cur[...] = h_nxt[...]

        @pl.when(l == n_layers - 1)
        def _emit():
            sep_ref[...] = sep_sum[...]
            sup = -sup_acc[0, 0] / (msk_acc[0, 0] / float(n_layers) + 1e-8)
            sup_ref[...] = sup * jnp.ones((1, 1), jnp.float32)


def kernel(x, adj, Wr0, Wi0, br0, bi0, conv_Wr, conv_Wi, conv_Wc,
           Wr1, Wi1, br1, bi1, labels, train_mask):
    n_nodes = adj.shape[0]
    f_in = x.shape[1]
    h_dim = Wr0.shape[1]
    n_cls = conv_Wc.shape[-1]
    n_layers = conv_Wr.shape[0]
    bn = next(b for b in (400, 200, 100, 40, 8, 4, 2, 1) if n_nodes % b == 0)
    nblk = n_nodes // bn
    nstream = next((k for k in (2, 1) if bn % k == 0 and (bn // k) % 8 == 0),
                   1)
    br = bn // nstream

    w0 = jnp.concatenate([Wr0, Wi0], axis=1)
    b0 = jnp.concatenate([br0, bi0])[None, :]
    ms = jnp.concatenate(
        [jnp.concatenate([conv_Wr, conv_Wi], axis=2),
         jnp.concatenate([-conv_Wi, conv_Wr], axis=2)], axis=1)
    mo = jnp.block([[Wr1, Wi1], [-Wi1, Wr1]])
    bo = jnp.concatenate([br1, bi1])[None, :]
    lab2 = labels.reshape(n_nodes, 1)
    msk2 = train_mask.astype(jnp.float32).reshape(n_nodes, 1)

    adj_specs = [
        pl.BlockSpec((br, n_nodes),
                     functools.partial(
                         lambda s, t: (nstream * (t % nblk) + s, 0), s))
        for s in range(nstream)
    ]
    const2 = lambda t: (0, 0)
    row2 = lambda t: (t % nblk, 0)
    lsel = lambda t: (t // nblk, 0, 0)
    outrow = lambda t: (jnp.where(t // nblk == n_layers - 1, t % nblk, 0), 0)
    in_specs = adj_specs + [
        pl.BlockSpec((n_nodes, f_in), const2),
        pl.BlockSpec((f_in, 2 * h_dim), const2),
        pl.BlockSpec((1, 2 * h_dim), const2),
        pl.BlockSpec((1, 2 * h_dim, 2 * h_dim), lsel),
        pl.BlockSpec((1, 2 * h_dim, n_cls), lsel),
        pl.BlockSpec((2 * h_dim, 2 * n_cls), const2),
        pl.BlockSpec((1, 2 * n_cls), const2),
        pl.BlockSpec((n_nodes, 1), const2),
        pl.BlockSpec((n_nodes, 1), const2),
    ]
    scalar_spec = pl.BlockSpec((1, 1), const2)
    scalar_shape = jax.ShapeDtypeStruct((1, 1), jnp.float32)
    nh_shape = jax.ShapeDtypeStruct((n_nodes, h_dim), jnp.float32)
    body = functools.partial(_fwd_body, n_layers=n_layers, bn=bn, nblk=nblk,
                             h_dim=h_dim, n_cls=n_cls, nstream=nstream)
    hr, hi, hang, hnrm, out_r, out_i, sep, sup = pl.pallas_call(
        body,
        grid=(n_layers * nblk,),
        in_specs=in_specs,
        out_specs=[
            pl.BlockSpec((bn, h_dim), outrow),
            pl.BlockSpec((bn, h_dim), outrow),
            pl.BlockSpec((bn, h_dim), outrow),
            pl.BlockSpec((bn, h_dim), outrow),
            pl.BlockSpec((bn, n_cls), outrow),
            pl.BlockSpec((bn, n_cls), outrow),
            scalar_spec, scalar_spec,
        ],
        out_shape=[
            nh_shape, nh_shape, nh_shape, nh_shape,
            jax.ShapeDtypeStruct((n_nodes, n_cls), jnp.float32),
            jax.ShapeDtypeStruct((n_nodes, n_cls), jnp.float32),
            scalar_shape, scalar_shape,
        ],
        compiler_params=pltpu.CompilerParams(
            vmem_limit_bytes=62 * 1024 * 1024),
        scratch_shapes=[
            pltpu.VMEM((n_nodes, 2 * h_dim), jnp.float32),
            pltpu.VMEM((n_nodes, 2 * h_dim), jnp.float32),
            pltpu.VMEM((n_cls, h_dim), jnp.float32),
            pltpu.VMEM((n_cls, h_dim), jnp.float32),
            pltpu.VMEM((1, 1), jnp.float32),
            pltpu.VMEM((1, 1), jnp.float32),
            pltpu.VMEM((1, 1), jnp.float32),
        ],
    )(*([adj] * nstream), x, w0, b0, ms, conv_Wc, mo, bo, lab2, msk2)

    return (hr, hi, out_r, out_i, hang, hnrm, sep[0, 0], sup[0, 0])
